# baseline (speedup 1.0000x reference)
import functools

import jax
import jax.numpy as jnp
from jax.experimental import pallas as pl
from jax.experimental.pallas import tpu as pltpu

B, T, D = 2, 2048, 1024
E, TOPK, H = 8, 2, 512
SH = H * TOPK
N = B * T
NE = E + TOPK
TM = 2048


def _fused_moe_body(rw_ref, rb_ref, x_ref, xb_ref, gw_ref, uw_ref, dw_ref, out_ref):
    e = pl.program_id(1)
    x = x_ref[...]

    logits = jnp.dot(x, rw_ref[...].T, preferred_element_type=jnp.float32)
    logits = logits + rb_ref[...]
    scores = jax.nn.softmax(logits, axis=-1)
    s1 = jnp.max(scores, axis=-1, keepdims=True)
    i1 = jnp.argmax(scores, axis=-1).reshape(TM, 1)
    cols = jax.lax.broadcasted_iota(jnp.int32, (TM, E), 1)
    masked = jnp.where(cols == i1, -jnp.inf, scores)
    s2 = jnp.max(masked, axis=-1, keepdims=True)
    i2 = jnp.argmax(masked, axis=-1).reshape(TM, 1)
    denom = s1 + s2
    w1 = s1 / denom
    w2 = s2 / denom
    w = jnp.where(i1 == e, w1, 0.0) + jnp.where(i2 == e, w2, 0.0)
    w = jnp.where(e >= E, 1.0, w)

    xb = xb_ref[...]
    g = jnp.dot(xb, gw_ref[0].T, preferred_element_type=jnp.float32)
    u = jnp.dot(xb, uw_ref[0].T, preferred_element_type=jnp.float32)
    h = ((g * jax.nn.sigmoid(g)) * u).astype(jnp.bfloat16)
    p = jnp.dot(h, dw_ref[0].T, preferred_element_type=jnp.float32)
    contrib = w * p

    @pl.when(e == 0)
    def _():
        out_ref[...] = contrib

    @pl.when(e != 0)
    def _():
        out_ref[...] += contrib


@jax.jit
def kernel(x, router_w, router_bias, gate_w, up_w, down_w, sg_w, su_w, sd_w):
    flat = x.reshape(N, D)
    flat_b = flat.astype(jnp.bfloat16)
    gw = jnp.concatenate([gate_w, sg_w.reshape(TOPK, H, D)], axis=0).astype(jnp.bfloat16)
    uw = jnp.concatenate([up_w, su_w.reshape(TOPK, H, D)], axis=0).astype(jnp.bfloat16)
    sd_split = jnp.stack([sd_w[:, :H], sd_w[:, H:]], axis=0)
    dw = jnp.concatenate([down_w, sd_split], axis=0).astype(jnp.bfloat16)
    rb = router_bias.reshape(1, E)

    grid = (N // TM, NE)
    out = pl.pallas_call(
        _fused_moe_body,
        grid=grid,
        in_specs=[
            pl.BlockSpec((E, D), lambda t, e: (0, 0)),
            pl.BlockSpec((1, E), lambda t, e: (0, 0)),
            pl.BlockSpec((TM, D), lambda t, e: (t, 0)),
            pl.BlockSpec((TM, D), lambda t, e: (t, 0)),
            pl.BlockSpec((1, H, D), lambda t, e: (e, 0, 0)),
            pl.BlockSpec((1, H, D), lambda t, e: (e, 0, 0)),
            pl.BlockSpec((1, D, H), lambda t, e: (e, 0, 0)),
        ],
        out_specs=pl.BlockSpec((TM, D), lambda t, e: (t, 0)),
        out_shape=jax.ShapeDtypeStruct((N, D), jnp.float32),
        compiler_params=pltpu.CompilerParams(
            dimension_semantics=("parallel", "arbitrary"),
        ),
    )(router_w, rb, flat, flat_b, gw, uw, dw)
    return out.reshape(B, T, D)


# speedup vs baseline: 1.0205x; 1.0205x over previous
import functools

import jax
import jax.numpy as jnp
from jax import lax
from jax.experimental import pallas as pl
from jax.experimental.pallas import tpu as pltpu
from jax.experimental.pallas import tpu_sc as plsc

B, T, D = 2, 2048, 1024
E, TOPK, H = 8, 2, 512
SH = H * TOPK
N = B * T
M = N * TOPK
TM2 = 128
BUF = M + E * TM2
NTILES = BUF // TM2
NW = 32
RPW = N // NW
TMS = 1024


def _cumsum_rows(a):
    rows = lax.broadcasted_iota(jnp.int32, a.shape, 0)
    k = 1
    while k < a.shape[0]:
        shifted = pltpu.roll(a, k, axis=0)
        a = a + jnp.where(rows >= k, shifted, 0.0)
        k *= 2
    return a


def _router_body(x_ref, rw_ref, rb_ref, pos_ref, w_ref, off_ref):
    x = x_ref[...]
    logits = jnp.dot(x, rw_ref[...].T, preferred_element_type=jnp.float32)
    logits = logits + rb_ref[...]
    scores = jax.nn.softmax(logits, axis=-1)
    s1 = jnp.max(scores, axis=-1, keepdims=True)
    i1 = jnp.argmax(scores, axis=-1).reshape(N, 1)
    cols = lax.broadcasted_iota(jnp.int32, (N, E), 1)
    masked = jnp.where(cols == i1, -jnp.inf, scores)
    s2 = jnp.max(masked, axis=-1, keepdims=True)
    i2 = jnp.argmax(masked, axis=-1).reshape(N, 1)
    denom = s1 + s2
    w1 = s1 / denom
    w2 = s2 / denom

    oh1 = (cols == i1).astype(jnp.float32)
    oh2 = (cols == i2).astype(jnp.float32)
    cs1 = _cumsum_rows(oh1)
    cs2 = _cumsum_rows(oh2)
    cnt1 = cs1[N - 1:N, :]
    counts = cnt1 + cs2[N - 1:N, :]
    padded = (counts.astype(jnp.int32) + (TM2 - 1)) & (-TM2)
    padded_f = padded.astype(jnp.float32)
    r8 = lax.broadcasted_iota(jnp.int32, (E, E), 0)
    c8 = lax.broadcasted_iota(jnp.int32, (E, E), 1)
    tril = (r8 < c8).astype(jnp.float32)
    off = jnp.dot(padded_f, tril, preferred_element_type=jnp.float32)

    rk1 = jnp.sum(oh1 * cs1, axis=1, keepdims=True) - 1.0
    rk2 = jnp.sum(oh2 * (cs2 + cnt1), axis=1, keepdims=True) - 1.0
    base1 = jnp.sum(oh1 * off, axis=1, keepdims=True)
    base2 = jnp.sum(oh2 * off, axis=1, keepdims=True)
    pos1 = base1 + rk1
    pos2 = base2 + rk2

    c128 = lax.broadcasted_iota(jnp.int32, (N, 128), 1)
    posmat = jnp.where(c128 == 0, pos1, jnp.where(c128 == 1, pos2, 0.0))
    pos_ref[...] = posmat.astype(jnp.int32)
    w_ref[...] = jnp.where(c128 == 0, w1, jnp.where(c128 == 1, w2, 0.0))
    spread = (lax.broadcasted_iota(jnp.int32, (E, 128), 0)
              == lax.broadcasted_iota(jnp.int32, (E, 128), 1)).astype(jnp.float32)
    off_ref[...] = jnp.dot(off, spread,
                           preferred_element_type=jnp.float32).astype(jnp.int32)


def _router_meta(flat, router_w, rb):
    return pl.pallas_call(
        _router_body,
        out_shape=[
            jax.ShapeDtypeStruct((N, 128), jnp.int32),
            jax.ShapeDtypeStruct((N, 128), jnp.float32),
            jax.ShapeDtypeStruct((1, 128), jnp.int32),
        ],
    )(flat, router_w, rb)


def _shared_body(x_ref, sg_ref, su_ref, sd_ref, out_ref):
    x = x_ref[...]
    g = jnp.dot(x, sg_ref[...].T, preferred_element_type=jnp.float32)
    u = jnp.dot(x, su_ref[...].T, preferred_element_type=jnp.float32)
    h = (g * jax.nn.sigmoid(g)) * u
    out_ref[...] = jnp.dot(h, sd_ref[...].T, preferred_element_type=jnp.float32)


def _shared_ffn(flat, sg_w, su_w, sd_w):
    return pl.pallas_call(
        _shared_body,
        grid=(N // TMS,),
        in_specs=[
            pl.BlockSpec((TMS, D), lambda t: (t, 0)),
            pl.BlockSpec((SH, D), lambda t: (0, 0)),
            pl.BlockSpec((SH, D), lambda t: (0, 0)),
            pl.BlockSpec((D, SH), lambda t: (0, 0)),
        ],
        out_specs=pl.BlockSpec((TMS, D), lambda t: (t, 0)),
        out_shape=jax.ShapeDtypeStruct((N, D), jnp.float32),
        compiler_params=pltpu.CompilerParams(
            dimension_semantics=("parallel",)),
    )(flat, sg_w, su_w, sd_w)


def _dispatch_body(x_hbm, posr_hbm, wrows_hbm, gx_hbm, wbuf_hbm,
                   rows_v, wrow_v, idx0_v, idx1_v, sem):
    cid = lax.axis_index("c")
    sid = lax.axis_index("s")
    wid = sid * 2 + cid
    for c in range(2):
        base = wid * RPW + c * 64
        pltpu.sync_copy(x_hbm.at[pl.ds(base, 64)], rows_v)
        pltpu.sync_copy(posr_hbm.at[wid, c, 0], idx0_v)
        pltpu.sync_copy(posr_hbm.at[wid, c, 1], idx1_v)
        d0 = pltpu.async_copy(rows_v, gx_hbm.at[idx0_v], sem)
        d1 = pltpu.async_copy(rows_v, gx_hbm.at[idx1_v], sem)
        d0.wait()
        d1.wait()
        pltpu.sync_copy(wrows_hbm.at[wid, c, 0], wrow_v)
        pltpu.async_copy(wrow_v, wbuf_hbm.at[idx0_v], sem).wait()
        pltpu.sync_copy(wrows_hbm.at[wid, c, 1], wrow_v)
        pltpu.async_copy(wrow_v, wbuf_hbm.at[idx1_v], sem).wait()


def _dispatch(flat, posr, wrows):
    mesh = plsc.VectorSubcoreMesh(core_axis_name="c", subcore_axis_name="s")
    return pl.kernel(
        _dispatch_body,
        out_type=[
            jax.ShapeDtypeStruct((BUF, D), jnp.float32),
            jax.ShapeDtypeStruct((BUF, 128), jnp.float32),
        ],
        mesh=mesh,
        scratch_types=[
            pltpu.VMEM((64, D), jnp.float32),
            pltpu.VMEM((64, 128), jnp.float32),
            pltpu.VMEM((64,), jnp.int32),
            pltpu.VMEM((64,), jnp.int32),
            pltpu.SemaphoreType.DMA,
        ],
    )(flat, posr, wrows)


def _group_gemm_body(te_ref, gx_ref, wb_ref, gw_ref, uw_ref, dw_ref, y_ref):
    xg = gx_ref[...]
    g = jnp.dot(xg, gw_ref[0].T, preferred_element_type=jnp.float32)
    u = jnp.dot(xg, uw_ref[0].T, preferred_element_type=jnp.float32)
    h = (g * jax.nn.sigmoid(g)) * u
    p = jnp.dot(h, dw_ref[0].T, preferred_element_type=jnp.float32)
    y_ref[...] = p * wb_ref[:, 0:1]


def _group_gemm(te, gx, wbuf, gate_w, up_w, down_w):
    grid_spec = pltpu.PrefetchScalarGridSpec(
        num_scalar_prefetch=1,
        grid=(NTILES,),
        in_specs=[
            pl.BlockSpec((TM2, D), lambda i, te: (i, 0)),
            pl.BlockSpec((TM2, 128), lambda i, te: (i, 0)),
            pl.BlockSpec((1, H, D), lambda i, te: (te[i], 0, 0)),
            pl.BlockSpec((1, H, D), lambda i, te: (te[i], 0, 0)),
            pl.BlockSpec((1, D, H), lambda i, te: (te[i], 0, 0)),
        ],
        out_specs=pl.BlockSpec((TM2, D), lambda i, te: (i, 0)),
    )
    return pl.pallas_call(
        _group_gemm_body,
        grid_spec=grid_spec,
        out_shape=jax.ShapeDtypeStruct((BUF, D), jnp.float32),
        compiler_params=pltpu.CompilerParams(
            dimension_semantics=("arbitrary",)),
    )(te, gx, wbuf, gate_w, up_w, down_w)


def _combine_body(shared_hbm, y_hbm, posq_hbm, out_hbm,
                  acc_v, y1_v, y2_v, idxa_v, idxb_v, sem):
    cid = lax.axis_index("c")
    sid = lax.axis_index("s")
    wid = sid * 2 + cid
    for c in range(4):
        base = wid * RPW + c * 32
        pltpu.sync_copy(posq_hbm.at[wid, c, 0], idxa_v)
        pltpu.sync_copy(posq_hbm.at[wid, c, 1], idxb_v)
        da = pltpu.async_copy(y_hbm.at[idxa_v], y1_v, sem)
        db = pltpu.async_copy(y_hbm.at[idxb_v], y2_v, sem)
        pltpu.sync_copy(shared_hbm.at[pl.ds(base, 32)], acc_v)
        da.wait()
        db.wait()

        def row_body(r, carry):
            for col in range(0, D, 16):
                s = pl.ds(col, 16)
                acc_v[r, s] = acc_v[r, s] + y1_v[r, s] + y2_v[r, s]
            return carry

        lax.fori_loop(0, 32, row_body, 0)
        pltpu.sync_copy(acc_v, out_hbm.at[pl.ds(base, 32)])


def _combine(shared, y, posq):
    mesh = plsc.VectorSubcoreMesh(core_axis_name="c", subcore_axis_name="s")
    return pl.kernel(
        _combine_body,
        out_type=jax.ShapeDtypeStruct((N, D), jnp.float32),
        mesh=mesh,
        scratch_types=[
            pltpu.VMEM((32, D), jnp.float32),
            pltpu.VMEM((32, D), jnp.float32),
            pltpu.VMEM((32, D), jnp.float32),
            pltpu.VMEM((32,), jnp.int32),
            pltpu.VMEM((32,), jnp.int32),
            pltpu.SemaphoreType.DMA,
        ],
    )(shared, y, posq)


@jax.jit
def kernel(x, router_w, router_bias, gate_w, up_w, down_w, sg_w, su_w, sd_w):
    flat = x.reshape(N, D)
    rb = router_bias.reshape(1, E)

    pos_out, w_out, off_out = _router_meta(flat, router_w, rb)

    pos_kn = pos_out[:, :TOPK].T
    w_kn = w_out[:, :TOPK].T
    posr = pos_kn.reshape(TOPK, NW, 2, 64).transpose(1, 2, 0, 3)
    posq = pos_kn.reshape(TOPK, NW, 4, 32).transpose(1, 2, 0, 3)
    wrows = jnp.broadcast_to(
        w_kn.reshape(TOPK, NW, 2, 64).transpose(1, 2, 0, 3)[..., None],
        (NW, 2, TOPK, 64, 128))
    off = off_out[0, :E]
    tile_start = jnp.arange(NTILES, dtype=jnp.int32) * TM2
    te = jnp.sum((off[None, :] <= tile_start[:, None]).astype(jnp.int32),
                 axis=1) - 1

    gx, wbuf = _dispatch(flat, posr, wrows)
    shared = _shared_ffn(flat, sg_w, su_w, sd_w)
    y = _group_gemm(te, gx, wbuf, gate_w, up_w, down_w)
    out = _combine(shared, y, posq)
    return out.reshape(B, T, D)
